# baseline (speedup 1.0000x reference)
import functools

import jax
import jax.numpy as jnp
from jax import lax
from jax.experimental import pallas as pl
from jax.experimental.pallas import tpu as pltpu
from jax.experimental.pallas import tpu_sc as plsc

_D = 64
_NC = 2
_NS = 16
_NW = _NC * _NS

_CHUNK = 512

_TB = 6400
_LNR = 1600


def _tc_table_pairs(tok_t):
    d, v = tok_t.shape
    nblk = -(-v // _TB)
    h = _TB // 2

    def body(x_ref, o_ref):
        xt = jnp.transpose(x_ref[...])
        o_ref[:, :d] = xt[:h, :]
        o_ref[:, d:] = xt[h:, :]

    return pl.pallas_call(
        body,
        grid=(nblk,),
        in_specs=[pl.BlockSpec((d, _TB), lambda i: (0, i))],
        out_specs=pl.BlockSpec((h, 2 * d), lambda i: (i, 0)),
        out_shape=jax.ShapeDtypeStruct((nblk * h, 2 * d), jnp.float32),
    )(tok_t)


def _sc_gather(ids2d, tok_lin):
    n128, lanes = ids2d.shape
    n = n128 * lanes
    rows_per_w = n // _NW
    chunks = rows_per_w // _CHUNK
    streams = _CHUNK // lanes
    idx_rows_per_w = rows_per_w // lanes
    mesh = plsc.VectorSubcoreMesh(core_axis_name="c", subcore_axis_name="s")

    @functools.partial(
        pl.kernel,
        mesh=mesh,
        out_type=jax.ShapeDtypeStruct((n, _D), jnp.float32),
        compiler_params=pltpu.CompilerParams(use_tc_tiling_on_sc=False),
        scratch_types=[
            pltpu.VMEM((_CHUNK // 128, 128), jnp.int32),
            pltpu.VMEM((_CHUNK, _D), jnp.float32),
            pltpu.SemaphoreType.DMA,
        ],
    )
    def k(ids_hbm, tok_hbm, out_hbm, idx_v, rows_v, sem):
        wid = lax.axis_index("s") * _NC + lax.axis_index("c")
        idx_base = wid * idx_rows_per_w
        row_base = wid * rows_per_w

        def body(c, carry):
            pltpu.sync_copy(ids_hbm.at[pl.ds(idx_base + c * streams,
                                             streams)], idx_v)
            cps = [
                pltpu.async_copy(tok_hbm.at[idx_v.at[j]],
                                 rows_v.at[pl.ds(j * lanes, lanes)], sem)
                for j in range(streams)
            ]
            for cp in cps:
                cp.wait()
            pltpu.sync_copy(rows_v,
                            out_hbm.at[pl.ds(row_base + c * _CHUNK, _CHUNK)])
            return carry

        lax.fori_loop(0, chunks, body, 0)

    return k(ids2d, tok_lin)


def _tc_ln_pairs(g_pairs, pos_big, scale, gamma2, beta2):
    npair = g_pairs.shape[0]
    nb = _LNR // 100

    def body(z_ref, p_ref, s_ref, gm_ref, bt_ref, o_ref):
        s = s_ref[0, 0]
        gm = gm_ref[...]
        bt = bt_ref[...]
        y = (z_ref[...] + p_ref[...]) * s

        def norm(x):
            mean = jnp.mean(x, axis=-1, keepdims=True)
            cen = x - mean
            var = jnp.mean(cen * cen, axis=-1, keepdims=True)
            return cen * lax.rsqrt(var + 1e-6) * gm + bt

        na = norm(y[:, :_D]).reshape(nb, 100, _D)
        nbv = norm(y[:, _D:]).reshape(nb, 100, _D)
        o_ref[...] = jnp.concatenate([na, nbv], axis=1).reshape(2 * _LNR, _D)

    return pl.pallas_call(
        body,
        grid=(npair // _LNR,),
        in_specs=[
            pl.BlockSpec((_LNR, 2 * _D), lambda i: (i, 0)),
            pl.BlockSpec((_LNR, 2 * _D), lambda i: (0, 0)),
            pl.BlockSpec(memory_space=pltpu.SMEM),
            pl.BlockSpec((1, _D), lambda i: (0, 0)),
            pl.BlockSpec((1, _D), lambda i: (0, 0)),
        ],
        out_specs=pl.BlockSpec((2 * _LNR, _D), lambda i: (i, 0)),
        out_shape=jax.ShapeDtypeStruct((2 * npair, _D), jnp.float32),
    )(g_pairs, pos_big, scale, gamma2, beta2)


def kernel(input_ids, tok, pos, embed_scale, gamma, beta):
    bc, tc = input_ids.shape
    half = tc // 2
    ids_lin = lax.optimization_barrier(
        input_ids.astype(jnp.int32).reshape(-1))
    ids_perm = ids_lin.reshape(bc, 2, half).transpose(0, 2, 1)
    hb = _TB // 2
    blk = ids_perm // _TB
    q = ids_perm - blk * _TB
    qm = jnp.where(q < hb, q, q - hb)
    ids_f = blk * _TB + 2 * qm + (q >= hb).astype(jnp.int32)
    ids2d = ids_f.reshape(-1, 128)

    tok_pairs = _tc_table_pairs(tok.T)
    tok_lin = tok_pairs.reshape(-1, _D)

    g = _sc_gather(ids2d, tok_lin)
    g_pairs = g.reshape(-1, 2 * _D)

    pos_lr = jnp.concatenate([pos[:half], pos[half:tc]], axis=1)
    pos_big = jnp.tile(pos_lr, (_LNR // half, 1))

    out = _tc_ln_pairs(g_pairs, pos_big,
                       embed_scale.reshape(1, 1).astype(jnp.float32),
                       gamma.reshape(1, _D), beta.reshape(1, _D))
    return out.reshape(bc, tc, _D)


# speedup vs baseline: 1.0794x; 1.0794x over previous
import functools

import jax
import jax.numpy as jnp
from jax import lax
from jax.experimental import pallas as pl
from jax.experimental.pallas import tpu as pltpu
from jax.experimental.pallas import tpu_sc as plsc

_D = 64
_NC = 2
_NS = 16
_NW = _NC * _NS

_CHUNK = 512

_TB = 12800
_LNR = 3200


def _tc_table_pairs(tok_t):
    d, v = tok_t.shape
    nblk = -(-v // _TB)
    h = _TB // 2

    def body(x_ref, o_ref):
        xt = jnp.transpose(x_ref[...])
        o_ref[:, :d] = xt[:h, :]
        o_ref[:, d:] = xt[h:, :]

    return pl.pallas_call(
        body,
        grid=(nblk,),
        in_specs=[pl.BlockSpec((d, _TB), lambda i: (0, i))],
        out_specs=pl.BlockSpec((h, 2 * d), lambda i: (i, 0)),
        out_shape=jax.ShapeDtypeStruct((nblk * h, 2 * d), jnp.float32),
    )(tok_t)


def _sc_gather(ids2d, tok_lin):
    n128, lanes = ids2d.shape
    n = n128 * lanes
    rows_per_w = n // _NW
    chunks = rows_per_w // _CHUNK
    streams = _CHUNK // lanes
    idx_rows_per_w = rows_per_w // lanes
    mesh = plsc.VectorSubcoreMesh(core_axis_name="c", subcore_axis_name="s")

    @functools.partial(
        pl.kernel,
        mesh=mesh,
        out_type=jax.ShapeDtypeStruct((n, _D), jnp.float32),
        compiler_params=pltpu.CompilerParams(use_tc_tiling_on_sc=False),
        scratch_types=[
            pltpu.VMEM((_CHUNK // 128, 128), jnp.int32),
            pltpu.VMEM((_CHUNK, _D), jnp.float32),
            pltpu.SemaphoreType.DMA,
        ],
    )
    def k(ids_hbm, tok_hbm, out_hbm, idx_v, rows_v, sem):
        wid = lax.axis_index("s") * _NC + lax.axis_index("c")
        idx_base = wid * idx_rows_per_w
        row_base = wid * rows_per_w

        def body(c, carry):
            pltpu.sync_copy(ids_hbm.at[pl.ds(idx_base + c * streams,
                                             streams)], idx_v)
            cps = [
                pltpu.async_copy(tok_hbm.at[idx_v.at[j]],
                                 rows_v.at[pl.ds(j * lanes, lanes)], sem)
                for j in range(streams)
            ]
            for cp in cps:
                cp.wait()
            pltpu.sync_copy(rows_v,
                            out_hbm.at[pl.ds(row_base + c * _CHUNK, _CHUNK)])
            return carry

        lax.fori_loop(0, chunks, body, 0)

    return k(ids2d, tok_lin)


def _tc_ln_pairs(g_pairs, pos_big, scale, gamma2, beta2):
    npair = g_pairs.shape[0]
    nb = _LNR // 100

    def body(z_ref, p_ref, s_ref, gm_ref, bt_ref, o_ref):
        s = s_ref[0, 0]
        gm = gm_ref[...]
        bt = bt_ref[...]
        y = (z_ref[...] + p_ref[...]) * s

        def norm(x):
            mean = jnp.mean(x, axis=-1, keepdims=True)
            cen = x - mean
            var = jnp.mean(cen * cen, axis=-1, keepdims=True)
            return cen * lax.rsqrt(var + 1e-6) * gm + bt

        na = norm(y[:, :_D]).reshape(nb, 100, _D)
        nbv = norm(y[:, _D:]).reshape(nb, 100, _D)
        o_ref[...] = jnp.concatenate([na, nbv], axis=1).reshape(2 * _LNR, _D)

    return pl.pallas_call(
        body,
        grid=(npair // _LNR,),
        in_specs=[
            pl.BlockSpec((_LNR, 2 * _D), lambda i: (i, 0)),
            pl.BlockSpec((_LNR, 2 * _D), lambda i: (0, 0)),
            pl.BlockSpec(memory_space=pltpu.SMEM),
            pl.BlockSpec((1, _D), lambda i: (0, 0)),
            pl.BlockSpec((1, _D), lambda i: (0, 0)),
        ],
        out_specs=pl.BlockSpec((2 * _LNR, _D), lambda i: (i, 0)),
        out_shape=jax.ShapeDtypeStruct((2 * npair, _D), jnp.float32),
    )(g_pairs, pos_big, scale, gamma2, beta2)


def kernel(input_ids, tok, pos, embed_scale, gamma, beta):
    bc, tc = input_ids.shape
    half = tc // 2
    ids_lin = input_ids.astype(jnp.int32).reshape(-1)
    ids_perm = ids_lin.reshape(bc, 2, half).transpose(0, 2, 1)
    hb = _TB // 2
    blk = ids_perm // _TB
    q = ids_perm - blk * _TB
    qm = jnp.where(q < hb, q, q - hb)
    ids_f = blk * _TB + 2 * qm + (q >= hb).astype(jnp.int32)
    ids2d = ids_f.reshape(-1, 128)

    tok_pairs = _tc_table_pairs(tok.T)
    tok_lin = tok_pairs.reshape(-1, _D)

    g = _sc_gather(ids2d, tok_lin)
    g_pairs = g.reshape(-1, 2 * _D)

    pos_lr = jnp.concatenate([pos[:half], pos[half:tc]], axis=1)
    pos_big = jnp.tile(pos_lr, (_LNR // half, 1))

    out = _tc_ln_pairs(g_pairs, pos_big,
                       embed_scale.reshape(1, 1).astype(jnp.float32),
                       gamma.reshape(1, _D), beta.reshape(1, _D))
    return out.reshape(bc, tc, _D)


# speedup vs baseline: 1.1310x; 1.0478x over previous
import functools

import jax
import jax.numpy as jnp
from jax import lax
from jax.experimental import pallas as pl
from jax.experimental.pallas import tpu as pltpu
from jax.experimental.pallas import tpu_sc as plsc

_D = 64
_NC = 2
_NS = 16
_NW = _NC * _NS

_CHUNK = 1024

_TB = 25600
_LNR = 6400


def _tc_table_pairs(tok_t):
    d, v = tok_t.shape
    nblk = -(-v // _TB)
    h = _TB // 2

    def body(x_ref, o_ref):
        xt = jnp.transpose(x_ref[...])
        o_ref[:, :d] = xt[:h, :]
        o_ref[:, d:] = xt[h:, :]

    return pl.pallas_call(
        body,
        grid=(nblk,),
        in_specs=[pl.BlockSpec((d, _TB), lambda i: (0, i))],
        out_specs=pl.BlockSpec((h, 2 * d), lambda i: (i, 0)),
        out_shape=jax.ShapeDtypeStruct((nblk * h, 2 * d), jnp.float32),
    )(tok_t)


def _sc_gather(ids2d, tok_lin):
    n128, lanes = ids2d.shape
    n = n128 * lanes
    rows_per_w = n // _NW
    chunks = rows_per_w // _CHUNK
    streams = _CHUNK // lanes
    idx_rows_per_w = rows_per_w // lanes
    mesh = plsc.VectorSubcoreMesh(core_axis_name="c", subcore_axis_name="s")

    @functools.partial(
        pl.kernel,
        mesh=mesh,
        out_type=jax.ShapeDtypeStruct((n, _D), jnp.float32),
        compiler_params=pltpu.CompilerParams(use_tc_tiling_on_sc=False),
        scratch_types=[
            pltpu.VMEM((_CHUNK // 128, 128), jnp.int32),
            pltpu.VMEM((_CHUNK, _D), jnp.float32),
            pltpu.SemaphoreType.DMA,
        ],
    )
    def k(ids_hbm, tok_hbm, out_hbm, idx_v, rows_v, sem):
        wid = lax.axis_index("s") * _NC + lax.axis_index("c")
        idx_base = wid * idx_rows_per_w
        row_base = wid * rows_per_w

        def body(c, carry):
            pltpu.sync_copy(ids_hbm.at[pl.ds(idx_base + c * streams,
                                             streams)], idx_v)
            cps = [
                pltpu.async_copy(tok_hbm.at[idx_v.at[j]],
                                 rows_v.at[pl.ds(j * lanes, lanes)], sem)
                for j in range(streams)
            ]
            for cp in cps:
                cp.wait()
            pltpu.sync_copy(rows_v,
                            out_hbm.at[pl.ds(row_base + c * _CHUNK, _CHUNK)])
            return carry

        lax.fori_loop(0, chunks, body, 0)

    return k(ids2d, tok_lin)


def _tc_ln_pairs(g_pairs, pos_big, scale, gamma2, beta2):
    npair = g_pairs.shape[0]
    nb = _LNR // 100

    def body(z_ref, p_ref, s_ref, gm_ref, bt_ref, o_ref):
        s = s_ref[0, 0]
        gm = gm_ref[...]
        bt = bt_ref[...]
        y = (z_ref[...] + p_ref[...]) * s

        def norm(x):
            mean = jnp.mean(x, axis=-1, keepdims=True)
            cen = x - mean
            var = jnp.mean(cen * cen, axis=-1, keepdims=True)
            return cen * lax.rsqrt(var + 1e-6) * gm + bt

        na = norm(y[:, :_D]).reshape(nb, 100, _D)
        nbv = norm(y[:, _D:]).reshape(nb, 100, _D)
        o_ref[...] = jnp.concatenate([na, nbv], axis=1).reshape(2 * _LNR, _D)

    return pl.pallas_call(
        body,
        grid=(npair // _LNR,),
        in_specs=[
            pl.BlockSpec((_LNR, 2 * _D), lambda i: (i, 0)),
            pl.BlockSpec((_LNR, 2 * _D), lambda i: (0, 0)),
            pl.BlockSpec(memory_space=pltpu.SMEM),
            pl.BlockSpec((1, _D), lambda i: (0, 0)),
            pl.BlockSpec((1, _D), lambda i: (0, 0)),
        ],
        out_specs=pl.BlockSpec((2 * _LNR, _D), lambda i: (i, 0)),
        out_shape=jax.ShapeDtypeStruct((2 * npair, _D), jnp.float32),
    )(g_pairs, pos_big, scale, gamma2, beta2)


def kernel(input_ids, tok, pos, embed_scale, gamma, beta):
    bc, tc = input_ids.shape
    half = tc // 2
    ids_lin = input_ids.astype(jnp.int32).reshape(-1)
    ids_perm = ids_lin.reshape(bc, 2, half).transpose(0, 2, 1)
    hb = _TB // 2
    blk = ids_perm // _TB
    q = ids_perm - blk * _TB
    qm = jnp.where(q < hb, q, q - hb)
    ids_f = blk * _TB + 2 * qm + (q >= hb).astype(jnp.int32)
    ids2d = ids_f.reshape(-1, 128)

    tok_pairs = _tc_table_pairs(tok.T)
    tok_lin = tok_pairs.reshape(-1, _D)

    g = _sc_gather(ids2d, tok_lin)
    g_pairs = g.reshape(-1, 2 * _D)

    pos_lr = jnp.concatenate([pos[:half], pos[half:tc]], axis=1)
    pos_big = jnp.tile(pos_lr, (_LNR // half, 1))

    out = _tc_ln_pairs(g_pairs, pos_big,
                       embed_scale.reshape(1, 1).astype(jnp.float32),
                       gamma.reshape(1, _D), beta.reshape(1, _D))
    return out.reshape(bc, tc, _D)


# speedup vs baseline: 1.1369x; 1.0052x over previous
import functools

import jax
import jax.numpy as jnp
from jax import lax
from jax.experimental import pallas as pl
from jax.experimental.pallas import tpu as pltpu
from jax.experimental.pallas import tpu_sc as plsc

_D = 64
_NC = 2
_NS = 16
_NW = _NC * _NS

_CHUNK = 1280

_TB = 25600
_LNR = 6400


def _tc_table_pairs(tok_t):
    d, v = tok_t.shape
    nblk = -(-v // _TB)
    h = _TB // 2

    def body(x_ref, o_ref):
        xt = jnp.transpose(x_ref[...])
        o_ref[:, :d] = xt[:h, :]
        o_ref[:, d:] = xt[h:, :]

    return pl.pallas_call(
        body,
        grid=(nblk,),
        in_specs=[pl.BlockSpec((d, _TB), lambda i: (0, i))],
        out_specs=pl.BlockSpec((h, 2 * d), lambda i: (i, 0)),
        out_shape=jax.ShapeDtypeStruct((nblk * h, 2 * d), jnp.float32),
    )(tok_t)


def _sc_gather(ids2d, tok_lin):
    n128, lanes = ids2d.shape
    n = n128 * lanes
    rows_per_w = n // _NW
    chunks = rows_per_w // _CHUNK
    streams = _CHUNK // lanes
    idx_rows_per_w = rows_per_w // lanes
    mesh = plsc.VectorSubcoreMesh(core_axis_name="c", subcore_axis_name="s")

    @functools.partial(
        pl.kernel,
        mesh=mesh,
        out_type=jax.ShapeDtypeStruct((n, _D), jnp.float32),
        compiler_params=pltpu.CompilerParams(use_tc_tiling_on_sc=False),
        scratch_types=[
            pltpu.VMEM((_CHUNK // 128, 128), jnp.int32),
            pltpu.VMEM((_CHUNK, _D), jnp.float32),
            pltpu.SemaphoreType.DMA,
        ],
    )
    def k(ids_hbm, tok_hbm, out_hbm, idx_v, rows_v, sem):
        wid = lax.axis_index("s") * _NC + lax.axis_index("c")
        idx_base = wid * idx_rows_per_w
        row_base = wid * rows_per_w

        def body(c, carry):
            pltpu.sync_copy(ids_hbm.at[pl.ds(idx_base + c * streams,
                                             streams)], idx_v)
            cps = [
                pltpu.async_copy(tok_hbm.at[idx_v.at[j]],
                                 rows_v.at[pl.ds(j * lanes, lanes)], sem)
                for j in range(streams)
            ]
            for cp in cps:
                cp.wait()
            pltpu.sync_copy(rows_v,
                            out_hbm.at[pl.ds(row_base + c * _CHUNK, _CHUNK)])
            return carry

        lax.fori_loop(0, chunks, body, 0)

    return k(ids2d, tok_lin)


def _tc_ln_pairs(g_pairs, pos_big, scale, gamma2, beta2):
    npair = g_pairs.shape[0]
    nb = _LNR // 100

    def body(z_ref, p_ref, s_ref, gm_ref, bt_ref, o_ref):
        s = s_ref[0, 0]
        gm = gm_ref[...]
        bt = bt_ref[...]
        y = (z_ref[...] + p_ref[...]) * s

        def norm(x):
            mean = jnp.mean(x, axis=-1, keepdims=True)
            cen = x - mean
            var = jnp.mean(cen * cen, axis=-1, keepdims=True)
            return cen * lax.rsqrt(var + 1e-6) * gm + bt

        na = norm(y[:, :_D]).reshape(nb, 100, _D)
        nbv = norm(y[:, _D:]).reshape(nb, 100, _D)
        o_ref[...] = jnp.concatenate([na, nbv], axis=1).reshape(2 * _LNR, _D)

    return pl.pallas_call(
        body,
        grid=(npair // _LNR,),
        in_specs=[
            pl.BlockSpec((_LNR, 2 * _D), lambda i: (i, 0)),
            pl.BlockSpec((_LNR, 2 * _D), lambda i: (0, 0)),
            pl.BlockSpec(memory_space=pltpu.SMEM),
            pl.BlockSpec((1, _D), lambda i: (0, 0)),
            pl.BlockSpec((1, _D), lambda i: (0, 0)),
        ],
        out_specs=pl.BlockSpec((2 * _LNR, _D), lambda i: (i, 0)),
        out_shape=jax.ShapeDtypeStruct((2 * npair, _D), jnp.float32),
    )(g_pairs, pos_big, scale, gamma2, beta2)


def kernel(input_ids, tok, pos, embed_scale, gamma, beta):
    bc, tc = input_ids.shape
    half = tc // 2
    ids_lin = input_ids.astype(jnp.int32).reshape(-1)
    ids_perm = ids_lin.reshape(bc, 2, half).transpose(0, 2, 1)
    hb = _TB // 2
    blk = ids_perm // _TB
    q = ids_perm - blk * _TB
    qm = jnp.where(q < hb, q, q - hb)
    ids_f = blk * _TB + 2 * qm + (q >= hb).astype(jnp.int32)
    ids2d = ids_f.reshape(-1, 128)

    tok_pairs = _tc_table_pairs(tok.T)
    tok_lin = tok_pairs.reshape(-1, _D)

    g = _sc_gather(ids2d, tok_lin)
    g_pairs = g.reshape(-1, 2 * _D)

    pos_lr = jnp.concatenate([pos[:half], pos[half:tc]], axis=1)
    pos_big = jnp.tile(pos_lr, (_LNR // half, 1))

    out = _tc_ln_pairs(g_pairs, pos_big,
                       embed_scale.reshape(1, 1).astype(jnp.float32),
                       gamma.reshape(1, _D), beta.reshape(1, _D))
    return out.reshape(bc, tc, _D)
